# baseline (speedup 1.0000x reference)
import functools

import jax
import jax.numpy as jnp
from jax import lax
from jax.experimental import pallas as pl
from jax.experimental.pallas import tpu as pltpu
from jax.experimental.pallas import tpu_sc as plsc

B_SIZE = 4096
DIM = 128
LANES = 16
NUM_CORES = 2
NUM_SUBCORES = 16
NUM_WORKERS = NUM_CORES * NUM_SUBCORES
PER_W = B_SIZE // NUM_WORKERS
GAMMA_C = 12.0
RHO_C = 0.4


def _make_sc_kernel():
    mesh = plsc.VectorSubcoreMesh(
        core_axis_name="c", subcore_axis_name="s",
        num_cores=NUM_CORES, num_subcores=NUM_SUBCORES)

    @functools.partial(
        pl.kernel,
        out_type=jax.ShapeDtypeStruct((B_SIZE,), jnp.float32),
        mesh=mesh,
        compiler_params=pltpu.CompilerParams(needs_layout_passes=False),
        scratch_types=[
            pltpu.VMEM((PER_W,), jnp.int32),
            pltpu.VMEM((PER_W,), jnp.int32),
            pltpu.VMEM((PER_W,), jnp.int32),
            pltpu.VMEM((PER_W, DIM), jnp.float32),
            pltpu.VMEM((PER_W, DIM), jnp.float32),
            pltpu.VMEM((PER_W, DIM), jnp.float32),
            pltpu.VMEM((PER_W, DIM), jnp.float32),
            pltpu.VMEM((PER_W, DIM), jnp.float32),
            pltpu.VMEM((PER_W,), jnp.float32),
            pltpu.SemaphoreType.DMA,
        ],
    )
    def kgfit_sc(heads_hbm, rels_hbm, tails_hbm, rel_tab, einit_tab, etext_tab,
                 out_hbm, h_v, r_v, t_v, hi_v, ht_v, ti_v, tt_v, rr_v,
                 score_v, sem):
        wid = lax.axis_index("s") * NUM_CORES + lax.axis_index("c")
        base = wid * PER_W

        pltpu.sync_copy(heads_hbm.at[pl.ds(base, PER_W)], h_v)
        pltpu.sync_copy(rels_hbm.at[pl.ds(base, PER_W)], r_v)
        pltpu.sync_copy(tails_hbm.at[pl.ds(base, PER_W)], t_v)

        d0 = pltpu.async_copy(einit_tab.at[h_v], hi_v, sem)
        d1 = pltpu.async_copy(etext_tab.at[h_v], ht_v, sem)
        d2 = pltpu.async_copy(einit_tab.at[t_v], ti_v, sem)
        d3 = pltpu.async_copy(etext_tab.at[t_v], tt_v, sem)
        d4 = pltpu.async_copy(rel_tab.at[r_v], rr_v, sem)
        d0.wait(); d1.wait(); d2.wait(); d3.wait(); d4.wait()

        lane = lax.iota(jnp.int32, LANES)

        def body(blk, carry):
            idx_s = blk * LANES + lane
            acc = jnp.zeros((LANES,), jnp.float32)
            for d in range(DIM):
                idx_d = jnp.full((LANES,), d, jnp.int32)
                hi = plsc.load_gather(hi_v, [idx_s, idx_d])
                ht = plsc.load_gather(ht_v, [idx_s, idx_d])
                ti = plsc.load_gather(ti_v, [idx_s, idx_d])
                tt = plsc.load_gather(tt_v, [idx_s, idx_d])
                rr = plsc.load_gather(rr_v, [idx_s, idx_d])
                v = (RHO_C * (hi - ti) + (1.0 - RHO_C) * (ht - tt) + rr)
                acc = acc + jnp.abs(v)
            score_v[pl.ds(blk * LANES, LANES)] = GAMMA_C - acc
            return carry

        lax.fori_loop(0, PER_W // LANES, body, 0)
        pltpu.sync_copy(score_v, out_hbm.at[pl.ds(base, PER_W)])

    return kgfit_sc


_KGFIT_SC = _make_sc_kernel()


@jax.jit
def kernel(sample, self_cluster_ids, neighbor_clusters_ids, parent_ids,
           relation_embedding, entity_embedding_init, entity_text_embeddings,
           cluster_embeddings):
    heads = sample[:, 0].astype(jnp.int32)
    rels = sample[:, 1].astype(jnp.int32)
    tails = sample[:, 2].astype(jnp.int32)
    scores = _KGFIT_SC(heads, rels, tails, relation_embedding,
                       entity_embedding_init, entity_text_embeddings)
    return scores.reshape(B_SIZE, 1)


# speedup vs baseline: 2.0656x; 2.0656x over previous
import functools

import jax
import jax.numpy as jnp
from jax import lax
from jax.experimental import pallas as pl
from jax.experimental.pallas import tpu as pltpu
from jax.experimental.pallas import tpu_sc as plsc

B_SIZE = 4096
DIM = 128
LANES = 16
NUM_CORES = 2
NUM_SUBCORES = 16
NUM_WORKERS = NUM_CORES * NUM_SUBCORES
PER_W = B_SIZE // NUM_WORKERS
GAMMA_C = 12.0
RHO_C = 0.4


def _make_sc_kernel():
    mesh = plsc.VectorSubcoreMesh(
        core_axis_name="c", subcore_axis_name="s",
        num_cores=NUM_CORES, num_subcores=NUM_SUBCORES)

    @functools.partial(
        pl.kernel,
        out_type=jax.ShapeDtypeStruct((B_SIZE,), jnp.float32),
        mesh=mesh,
        compiler_params=pltpu.CompilerParams(needs_layout_passes=False),
        scratch_types=[
            pltpu.VMEM((PER_W,), jnp.int32),
            pltpu.VMEM((PER_W,), jnp.int32),
            pltpu.VMEM((PER_W,), jnp.int32),
            pltpu.VMEM((PER_W, DIM), jnp.float32),
            pltpu.VMEM((PER_W, DIM), jnp.float32),
            pltpu.VMEM((PER_W, DIM), jnp.float32),
            pltpu.VMEM((PER_W, DIM), jnp.float32),
            pltpu.VMEM((PER_W, DIM), jnp.float32),
            pltpu.VMEM((PER_W,), jnp.float32),
            pltpu.SemaphoreType.DMA,
        ],
    )
    def kgfit_sc(heads_hbm, rels_hbm, tails_hbm, rel_tab, einit_tab, etext_tab,
                 out_hbm, h_v, r_v, t_v, hi_v, ht_v, ti_v, tt_v, rr_v,
                 score_v, sem):
        wid = lax.axis_index("s") * NUM_CORES + lax.axis_index("c")
        base = wid * PER_W

        pltpu.sync_copy(heads_hbm.at[pl.ds(base, PER_W)], h_v)
        pltpu.sync_copy(rels_hbm.at[pl.ds(base, PER_W)], r_v)
        pltpu.sync_copy(tails_hbm.at[pl.ds(base, PER_W)], t_v)

        d0 = pltpu.async_copy(einit_tab.at[h_v], hi_v, sem)
        d1 = pltpu.async_copy(etext_tab.at[h_v], ht_v, sem)
        d2 = pltpu.async_copy(einit_tab.at[t_v], ti_v, sem)
        d3 = pltpu.async_copy(etext_tab.at[t_v], tt_v, sem)
        d4 = pltpu.async_copy(rel_tab.at[r_v], rr_v, sem)
        d0.wait(); d1.wait(); d2.wait(); d3.wait(); d4.wait()

        lane = lax.iota(jnp.int32, LANES)

        def body(blk, carry):
            score = jnp.zeros((LANES,), jnp.float32)
            for k in range(LANES):
                i = blk * LANES + k
                acc = jnp.zeros((LANES,), jnp.float32)
                for j in range(DIM // LANES):
                    sl = pl.ds(j * LANES, LANES)
                    v = (RHO_C * (hi_v[i, sl] - ti_v[i, sl])
                         + (1.0 - RHO_C) * (ht_v[i, sl] - tt_v[i, sl])
                         + rr_v[i, sl])
                    acc = acc + jnp.abs(v)
                score = jnp.where(lane == k, GAMMA_C - jnp.sum(acc), score)
            score_v[pl.ds(blk * LANES, LANES)] = score
            return carry

        lax.fori_loop(0, PER_W // LANES, body, 0)
        pltpu.sync_copy(score_v, out_hbm.at[pl.ds(base, PER_W)])

    return kgfit_sc


_KGFIT_SC = _make_sc_kernel()


@jax.jit
def kernel(sample, self_cluster_ids, neighbor_clusters_ids, parent_ids,
           relation_embedding, entity_embedding_init, entity_text_embeddings,
           cluster_embeddings):
    heads = sample[:, 0].astype(jnp.int32)
    rels = sample[:, 1].astype(jnp.int32)
    tails = sample[:, 2].astype(jnp.int32)
    scores = _KGFIT_SC(heads, rels, tails, relation_embedding,
                       entity_embedding_init, entity_text_embeddings)
    return scores.reshape(B_SIZE, 1)
